# baseline (speedup 1.0000x reference)
import functools

import jax
import jax.numpy as jnp
from jax import lax
from jax.experimental import pallas as pl
from jax.experimental.pallas import tpu as pltpu
from jax.experimental.pallas import tpu_sc as plsc

VOCAB = 100000
DIM = 64
BATCH = 16384
NC = 2
NS = 16
NW = NC * NS
LANES = 16
CHUNK = 2048
NCHUNK = BATCH // CHUNK
UNROLL = 16
TCB = 2048

_params = pltpu.CompilerParams(
    needs_layout_passes=False, use_tc_tiling_on_sc=True)
_mesh = plsc.VectorSubcoreMesh(core_axis_name="c", subcore_axis_name="s")


def _gather_body(ctab, xtab, cidx, xidx, gc, gx,
                 row_v, idx_v, buf_v, sem):
    wid = lax.axis_index("s") * NC + lax.axis_index("c")

    def do_feature(tab, out, j):
        pltpu.sync_copy(tab.at[j], row_v)

        groups_per_chunk = CHUNK // LANES
        prev = None
        for k in range(NCHUNK):
            half = k % 2

            @plsc.parallel_loop(k * groups_per_chunk,
                                (k + 1) * groups_per_chunk,
                                unroll=UNROLL)
            def block(g):
                ids = idx_v[pl.ds(g * LANES, LANES)]
                buf_v[half, pl.ds(g * LANES - k * CHUNK, LANES)] = (
                    plsc.load_gather(row_v, [ids]))
            if prev is not None:
                prev.wait()
            prev = pltpu.async_copy(
                buf_v.at[half], out.at[j, pl.ds(k * CHUNK, CHUNK)], sem)
        prev.wait()

    @pl.when(lax.axis_index("c") == 0)
    def _():
        sid = lax.axis_index("s")
        pltpu.sync_copy(cidx, idx_v)
        do_feature(ctab, gc, sid)
        do_feature(ctab, gc, sid + 16)
        do_feature(ctab, gc, sid + 32)
        do_feature(ctab, gc, sid + 48)
        pltpu.sync_copy(xidx, idx_v)
        do_feature(xtab, gx, sid)
        do_feature(xtab, gx, sid + 16)
        do_feature(xtab, gx, sid + 32)
        do_feature(xtab, gx, sid + 48)


def _tc_dot_body(gc_ref, gx_ref, out_ref):
    out_ref[...] = jnp.sum(gc_ref[...] * gx_ref[...], axis=0)


@jax.jit
def kernel(center_words, context_words, center_table, context_table):
    cidx = center_words.astype(jnp.int32)
    xidx = context_words.astype(jnp.int32)
    ctab_t = center_table.T
    xtab_t = context_table.T

    gather = pl.kernel(
        _gather_body,
        mesh=_mesh,
        out_type=(
            jax.ShapeDtypeStruct((DIM, BATCH), jnp.float32),
            jax.ShapeDtypeStruct((DIM, BATCH), jnp.float32),
        ),
        scratch_types=[
            pltpu.VMEM((VOCAB,), jnp.float32),
            pltpu.VMEM((BATCH,), jnp.int32),
            pltpu.VMEM((2, CHUNK), jnp.float32),
            pltpu.SemaphoreType.DMA,
        ],
        compiler_params=_params,
    )
    gc, gx = gather(ctab_t, xtab_t, cidx, xidx)

    dot = pl.pallas_call(
        _tc_dot_body,
        grid=(BATCH // TCB,),
        in_specs=[
            pl.BlockSpec((DIM, TCB), lambda i: (0, i)),
            pl.BlockSpec((DIM, TCB), lambda i: (0, i)),
        ],
        out_specs=pl.BlockSpec((TCB,), lambda i: (i,)),
        out_shape=jax.ShapeDtypeStruct((BATCH,), jnp.float32),
    )
    return dot(gc, gx)


# speedup vs baseline: 1.4735x; 1.4735x over previous
import functools

import jax
import jax.numpy as jnp
from jax import lax
from jax.experimental import pallas as pl
from jax.experimental.pallas import tpu as pltpu
from jax.experimental.pallas import tpu_sc as plsc

VOCAB = 100000
DIM = 64
BATCH = 16384
NC = 2
NS = 16
NW = NC * NS
LANES = 16
CHUNK = 4096
NCHUNK = BATCH // CHUNK
UNROLL = 16
TCB = 2048

_params = pltpu.CompilerParams(
    needs_layout_passes=False, use_tc_tiling_on_sc=True)
_mesh = plsc.VectorSubcoreMesh(core_axis_name="c", subcore_axis_name="s")


def _gather_body(ctab, xtab, cidx, xidx, gc, gx,
                 row_v, idx_v, buf_v, sem):
    wid = lax.axis_index("s") * NC + lax.axis_index("c")

    def do_feature(tab, out, j):
        pltpu.sync_copy(tab.at[j], row_v)

        groups_per_chunk = CHUNK // LANES
        prev = None
        for k in range(NCHUNK):
            half = k % 2

            @plsc.parallel_loop(k * groups_per_chunk,
                                (k + 1) * groups_per_chunk,
                                unroll=UNROLL)
            def block(g):
                ids = idx_v[pl.ds(g * LANES, LANES)]
                buf_v[half, pl.ds(g * LANES - k * CHUNK, LANES)] = (
                    plsc.load_gather(row_v, [ids]))
            if prev is not None:
                prev.wait()
            prev = pltpu.async_copy(
                buf_v.at[half], out.at[j, pl.ds(k * CHUNK, CHUNK)], sem)
        prev.wait()

    pltpu.sync_copy(cidx, idx_v)
    do_feature(ctab, gc, wid)
    do_feature(ctab, gc, wid + NW)
    pltpu.sync_copy(xidx, idx_v)
    do_feature(xtab, gx, wid)
    do_feature(xtab, gx, wid + NW)


def _tc_dot_body(gc_ref, gx_ref, out_ref):
    out_ref[...] = jnp.sum(gc_ref[...] * gx_ref[...], axis=0)


@jax.jit
def kernel(center_words, context_words, center_table, context_table):
    cidx = center_words.astype(jnp.int32)
    xidx = context_words.astype(jnp.int32)
    ctab_t = center_table.T
    xtab_t = context_table.T

    gather = pl.kernel(
        _gather_body,
        mesh=_mesh,
        out_type=(
            jax.ShapeDtypeStruct((DIM, BATCH), jnp.float32),
            jax.ShapeDtypeStruct((DIM, BATCH), jnp.float32),
        ),
        scratch_types=[
            pltpu.VMEM((VOCAB,), jnp.float32),
            pltpu.VMEM((BATCH,), jnp.int32),
            pltpu.VMEM((2, CHUNK), jnp.float32),
            pltpu.SemaphoreType.DMA,
        ],
        compiler_params=_params,
    )
    gc, gx = gather(ctab_t, xtab_t, cidx, xidx)

    dot = pl.pallas_call(
        _tc_dot_body,
        grid=(BATCH // TCB,),
        in_specs=[
            pl.BlockSpec((DIM, TCB), lambda i: (0, i)),
            pl.BlockSpec((DIM, TCB), lambda i: (0, i)),
        ],
        out_specs=pl.BlockSpec((TCB,), lambda i: (i,)),
        out_shape=jax.ShapeDtypeStruct((BATCH,), jnp.float32),
    )
    return dot(gc, gx)
